# baseline (speedup 1.0000x reference)
import functools

import jax
import jax.numpy as jnp
from jax import lax
from jax.experimental import pallas as pl
from jax.experimental.pallas import tpu as pltpu
from jax.experimental.pallas import tpu_sc as plsc

LANES = 16


def kernel(x, token_table, pos_table):
    B, L = x.shape
    V, D = token_table.shape

    info = plsc.get_sparse_core_info()
    NC, NS = info.num_cores, info.num_subcores
    NW = NC * NS

    rows_per_w = B // NW
    CHUNK_ROWS = 4
    CHUNK_TOK = CHUNK_ROWS * L
    N_CHUNKS = rows_per_w // CHUNK_ROWS
    G = 8
    IDX_PER_G = CHUNK_TOK // G
    DG = D // LANES

    x_flat = x.reshape(NW, N_CHUNKS, G, IDX_PER_G).astype(jnp.int32)

    mesh = plsc.VectorSubcoreMesh(core_axis_name="c", subcore_axis_name="s")

    @functools.partial(
        pl.kernel,
        mesh=mesh,
        compiler_params=pltpu.CompilerParams(use_tc_tiling_on_sc=False),
        out_type=jax.ShapeDtypeStruct((B * L, D), jnp.float32),
        scratch_types=[
            pltpu.VMEM((L, D), jnp.float32),
            pltpu.VMEM((G, IDX_PER_G), jnp.int32),
            pltpu.VMEM((CHUNK_TOK, D), jnp.float32),
            pltpu.SemaphoreType.DMA,
        ],
    )
    def emb_kernel(x_hbm, tok_hbm, pos_hbm, out_hbm, pos_v, idx_v, rows_v, sem):
        wid = lax.axis_index("s") * NC + lax.axis_index("c")
        pltpu.sync_copy(pos_hbm, pos_v)
        w_base = wid * (rows_per_w * L)

        def chunk_body(c, carry):
            tok_base = w_base + c * CHUNK_TOK
            pltpu.sync_copy(x_hbm.at[wid, c], idx_v)
            for j in range(G):
                pltpu.async_copy(
                    tok_hbm.at[idx_v.at[j]],
                    rows_v.at[pl.ds(j * IDX_PER_G, IDX_PER_G)],
                    sem,
                ).wait()

            def pos_body(p, carry2):
                for g in range(DG):
                    pv = pos_v[p, pl.ds(g * LANES, LANES)]
                    for rep in range(CHUNK_ROWS):
                        t = rep * L + p
                        rows_v[t, pl.ds(g * LANES, LANES)] = (
                            rows_v[t, pl.ds(g * LANES, LANES)] + pv
                        )
                return carry2

            lax.fori_loop(0, L, pos_body, 0)
            pltpu.sync_copy(rows_v, out_hbm.at[pl.ds(tok_base, CHUNK_TOK)])
            return carry

        lax.fori_loop(0, N_CHUNKS, chunk_body, 0)

    out = emb_kernel(x_flat, token_table, pos_table)
    return out.reshape(B, L, D)


# speedup vs baseline: 1.1904x; 1.1904x over previous
import functools

import jax
import jax.numpy as jnp
from jax import lax
from jax.experimental import pallas as pl
from jax.experimental.pallas import tpu as pltpu
from jax.experimental.pallas import tpu_sc as plsc

LANES = 16


def kernel(x, token_table, pos_table):
    B, L = x.shape
    V, D = token_table.shape

    info = plsc.get_sparse_core_info()
    NC, NS = info.num_cores, info.num_subcores
    NW = NC * NS

    tok_per_w = (B * L) // NW
    CHUNK = L
    N_CHUNKS = tok_per_w // CHUNK
    IDX_PER_G = 100
    G = CHUNK // IDX_PER_G
    NB = 4
    DG = D // LANES
    VECS = (CHUNK * D) // LANES

    x_flat = x.reshape(NW, N_CHUNKS * G, IDX_PER_G).astype(jnp.int32)

    mesh = plsc.VectorSubcoreMesh(core_axis_name="c", subcore_axis_name="s")

    @functools.partial(
        pl.kernel,
        mesh=mesh,
        compiler_params=pltpu.CompilerParams(use_tc_tiling_on_sc=False),
        out_type=jax.ShapeDtypeStruct((B * L, D), jnp.float32),
        scratch_types=[
            pltpu.VMEM((L, D), jnp.float32),
            pltpu.VMEM((N_CHUNKS * G, IDX_PER_G), jnp.int32),
            [pltpu.VMEM((CHUNK, D), jnp.float32) for _ in range(NB)],
            [pltpu.SemaphoreType.DMA for _ in range(NB)],
            [pltpu.SemaphoreType.DMA for _ in range(NB)],
        ],
    )
    def emb_kernel(x_hbm, tok_hbm, pos_hbm, out_hbm, pos_v, idx_v, rows, gsem, wsem):
        wid = lax.axis_index("s") * NC + lax.axis_index("c")
        pltpu.sync_copy(pos_hbm, pos_v)
        pltpu.sync_copy(x_hbm.at[wid], idx_v)
        w_base = wid * tok_per_w

        def fire_gather(c, b):
            for j in range(G):
                pltpu.async_copy(
                    tok_hbm.at[idx_v.at[c * G + j]],
                    rows[b].at[pl.ds(j * IDX_PER_G, IDX_PER_G)],
                    gsem[b],
                )

        def drain_gather(b):
            pltpu.make_async_copy(
                tok_hbm.at[pl.ds(0, CHUNK)], rows[b], gsem[b]
            ).wait()

        def drain_write(b):
            pltpu.make_async_copy(
                rows[b], out_hbm.at[pl.ds(0, CHUNK)], wsem[b]
            ).wait()

        fire_gather(0, 0)
        fire_gather(1, 1)

        def quad_body(c4, carry):
            for b in range(NB):
                c = c4 * NB + b
                bn = (b + 2) % NB

                @pl.when(c + 2 < N_CHUNKS)
                def _():
                    @pl.when(c >= 2)
                    def _():
                        drain_write(bn)

                    fire_gather(c + 2, bn)

                drain_gather(b)

                def add_row(r, carry2):
                    for g in range(DG):
                        sl = pl.ds(g * LANES, LANES)
                        rows[b][r, sl] = rows[b][r, sl] + pos_v[r, sl]
                    return carry2

                lax.fori_loop(0, CHUNK, add_row, 0)
                pltpu.async_copy(
                    rows[b],
                    out_hbm.at[pl.ds(w_base + c * CHUNK, CHUNK)],
                    wsem[b],
                )
            return carry

        lax.fori_loop(0, N_CHUNKS // NB, quad_body, 0)
        for b in range(NB):
            drain_write(b)

    out = emb_kernel(x_flat, token_table, pos_table)
    return out.reshape(B, L, D)


# speedup vs baseline: 1.1922x; 1.0015x over previous
import functools

import jax
import jax.numpy as jnp
from jax import lax
from jax.experimental import pallas as pl
from jax.experimental.pallas import tpu as pltpu
from jax.experimental.pallas import tpu_sc as plsc

LANES = 16


def kernel(x, token_table, pos_table):
    B, L = x.shape
    V, D = token_table.shape

    info = plsc.get_sparse_core_info()
    NC, NS = info.num_cores, info.num_subcores
    NW = NC * NS

    rows_per_w = B // NW
    CHUNK = L
    N_CHUNKS = rows_per_w
    G_SPLIT = (120, 80)
    NB = 4
    DG = D // LANES

    mesh = plsc.VectorSubcoreMesh(core_axis_name="c", subcore_axis_name="s")

    @functools.partial(
        pl.kernel,
        mesh=mesh,
        compiler_params=pltpu.CompilerParams(use_tc_tiling_on_sc=False),
        out_type=jax.ShapeDtypeStruct((B, L, D), jnp.float32),
        scratch_types=[
            pltpu.VMEM((L, D), jnp.float32),
            pltpu.VMEM((rows_per_w, L), jnp.int32),
            [pltpu.VMEM((CHUNK, D), jnp.float32) for _ in range(NB)],
            [pltpu.SemaphoreType.DMA for _ in range(NB)],
            [pltpu.SemaphoreType.DMA for _ in range(NB)],
        ],
    )
    def emb_kernel(x_hbm, tok_hbm, pos_hbm, out_hbm, pos_v, idx_v, rows, gsem, wsem):
        wid = lax.axis_index("s") * NC + lax.axis_index("c")
        seq0 = wid * rows_per_w
        pltpu.sync_copy(pos_hbm, pos_v)
        pltpu.sync_copy(x_hbm.at[pl.ds(seq0, rows_per_w)], idx_v)

        def fire_gather(c, b):
            off = 0
            for n in G_SPLIT:
                pltpu.async_copy(
                    tok_hbm.at[idx_v.at[c, pl.ds(off, n)]],
                    rows[b].at[pl.ds(off, n)],
                    gsem[b],
                )
                off += n

        def drain_gather(b):
            pltpu.make_async_copy(
                tok_hbm.at[pl.ds(0, CHUNK)], rows[b], gsem[b]
            ).wait()

        def drain_write(b):
            pltpu.make_async_copy(rows[b], out_hbm.at[0], wsem[b]).wait()

        fire_gather(0, 0)
        fire_gather(1, 1)

        def quad_body(c4, carry):
            for b in range(NB):
                c = c4 * NB + b
                bn = (b + 2) % NB

                @pl.when(c + 2 < N_CHUNKS)
                def _():
                    @pl.when(c >= 2)
                    def _():
                        drain_write(bn)

                    fire_gather(c + 2, bn)

                drain_gather(b)

                def add_row(r, carry2):
                    for g in range(DG):
                        sl = pl.ds(g * LANES, LANES)
                        rows[b][r, sl] = rows[b][r, sl] + pos_v[r, sl]
                    return carry2

                lax.fori_loop(0, CHUNK, add_row, 0)
                pltpu.async_copy(rows[b], out_hbm.at[seq0 + c], wsem[b])
            return carry

        lax.fori_loop(0, N_CHUNKS // NB, quad_body, 0)
        for b in range(NB):
            drain_write(b)

    return emb_kernel(x, token_table, pos_table)
